# baseline (speedup 1.0000x reference)
import functools

import jax
import jax.numpy as jnp
from jax import lax
from jax.experimental import pallas as pl
from jax.experimental.pallas import tpu as pltpu
from jax.experimental.pallas import tpu_sc as plsc

_MAX_ID = 128
_L = 16
_NW = 32


def _sc_partials(emb, ids):
    B, F, C, H, W = emb.shape
    BF = B * F
    tiles_per_frame = _NW // BF
    rows = H // tiles_per_frame
    nbuf = 2
    rsub = 2
    n_sub = rows // rsub
    grow = W // _L
    gsub = rsub * grow
    acc_rows = C + 1
    seg = _L * _MAX_ID

    mesh = plsc.VectorSubcoreMesh(core_axis_name="c", subcore_axis_name="s")

    @functools.partial(
        pl.kernel,
        out_type=jax.ShapeDtypeStruct((_NW, acc_rows, _MAX_ID), jnp.float32),
        mesh=mesh,
        compiler_params=pltpu.CompilerParams(needs_layout_passes=False),
        scratch_types=[
            pltpu.VMEM((rows, W), jnp.int32),
            pltpu.VMEM((nbuf, C, rsub, W), jnp.float32),
            pltpu.VMEM((acc_rows * seg,), jnp.float32),
            pltpu.VMEM((acc_rows, _MAX_ID), jnp.float32),
            pltpu.SemaphoreType.DMA,
            pltpu.SemaphoreType.DMA,
            pltpu.SemaphoreType.DMA,
        ],
    )
    def sc_kernel(emb_hbm, ids_hbm, out_hbm, ids_v, val_v, acc_v, stage_v,
                  sem_ids, sem_a, sem_b):
        wid = lax.axis_index("s") * 2 + lax.axis_index("c")
        frame = wid // tiles_per_frame
        b = frame // F
        f = frame % F
        r0 = (wid % tiles_per_frame) * rows

        pltpu.make_async_copy(
            ids_hbm.at[b, f, pl.ds(r0, rows), :], ids_v, sem_ids).start()
        sems = (sem_a, sem_b)
        for buf in range(nbuf - 1):
            pltpu.make_async_copy(
                emb_hbm.at[b, f, :, pl.ds(r0 + buf * rsub, rsub), :],
                val_v.at[buf], sems[buf]).start()

        zero = jnp.zeros((_L,), jnp.float32)

        def zero_body(i, carry):
            acc_v[pl.ds(i * _L, _L)] = zero
            return carry

        lax.fori_loop(0, acc_rows * seg // _L, zero_body, 0, unroll=8)
        pltpu.make_async_copy(
            ids_hbm.at[b, f, pl.ds(r0, rows), :], ids_v, sem_ids).wait()

        lane_off = lax.iota(jnp.int32, _L) * _MAX_ID
        ones = jnp.ones((_L,), jnp.float32)

        def sub_body(i, carry):
            for buf in range(nbuf):
                s = nbuf * i + buf
                nxt = (buf + nbuf - 1) % nbuf

                @pl.when(s + nbuf - 1 < n_sub)
                def _prefetch(s=s, nxt=nxt):
                    pltpu.make_async_copy(
                        emb_hbm.at[b, f, :,
                                   pl.ds(r0 + (s + nbuf - 1) * rsub, rsub),
                                   :],
                        val_v.at[nxt], sems[nxt]).start()

                pltpu.make_async_copy(
                    emb_hbm.at[b, f, :, pl.ds(r0 + s * rsub, rsub), :],
                    val_v.at[buf], sems[buf]).wait()

                @plsc.parallel_loop(0, gsub, unroll=2)
                def gbody(g, s=s, buf=buf):
                    gr = g // grow
                    col = (g % grow) * _L
                    idx = ids_v[s * rsub + gr, pl.ds(col, _L)] + lane_off
                    for c in range(C):
                        plsc.addupdate_scatter(
                            acc_v, [idx + c * seg],
                            val_v[buf, c, gr, pl.ds(col, _L)])
                    plsc.addupdate_scatter(acc_v, [idx + C * seg], ones)
            return carry

        lax.fori_loop(0, n_sub // nbuf, sub_body, 0)

        def red_body(r, carry):
            base = r * seg
            for blk in range(_MAX_ID // _L):
                o = blk * _L
                vs = [acc_v[pl.ds(base + l * _MAX_ID + o, _L)]
                      for l in range(_L)]
                while len(vs) > 1:
                    vs = [vs[i] + vs[i + 1] for i in range(0, len(vs), 2)]
                stage_v[r, pl.ds(o, _L)] = vs[0]
            return carry

        lax.fori_loop(0, acc_rows, red_body, 0)

        pltpu.sync_copy(stage_v, out_hbm.at[wid])

    return sc_kernel(emb, ids)


def _tc_finalize(partials, B, F, C):
    tiles_per_frame = _NW // (B * F)

    def tc_kernel(p_ref, o_ref):
        p = p_ref[...]
        p = p.reshape(B * F, tiles_per_frame, C + 1, _MAX_ID).sum(axis=1)
        sums = p[:, :C, :].reshape(B, F, C, _MAX_ID)
        counts = p[:, C, :].reshape(B, F, _MAX_ID)
        means = sums / jnp.maximum(counts, 1.0)[:, :, None, :]
        idpos = lax.broadcasted_iota(jnp.int32, (B, F, _MAX_ID), 2) > 0
        present = (counts > 0.0) & idpos
        common = present[:, :-1] & present[:, 1:]
        d = means[:, 1:] - means[:, :-1]
        dist = jnp.sum(d * d, axis=2)
        total = jnp.sum(jnp.where(common, dist, 0.0))
        valid = jnp.sum(common.astype(jnp.float32))
        o_ref[0, 0] = jnp.where(valid > 0.0,
                                total / jnp.maximum(valid, 1.0),
                                jnp.float32(0.0))

    return pl.pallas_call(
        tc_kernel,
        out_shape=jax.ShapeDtypeStruct((1, 1), jnp.float32),
        out_specs=pl.BlockSpec(memory_space=pltpu.SMEM),
    )(partials)


def kernel(embeddings, track_ids):
    B, F, C, H, W = embeddings.shape
    ids = track_ids.reshape(B, F, H, W).astype(jnp.int32)
    partials = _sc_partials(embeddings, ids)
    return _tc_finalize(partials, B, F, C)[0, 0]


# speedup vs baseline: 1.0280x; 1.0280x over previous
import functools

import jax
import jax.numpy as jnp
from jax import lax
from jax.experimental import pallas as pl
from jax.experimental.pallas import tpu as pltpu
from jax.experimental.pallas import tpu_sc as plsc

_MAX_ID = 128
_L = 16
_NW = 32


def _sc_partials(emb, ids):
    B, F, C, H, W = emb.shape
    BF = B * F
    tiles_per_frame = _NW // BF
    rows = H // tiles_per_frame
    nbuf = 2
    rsub = 2
    n_sub = rows // rsub
    grow = W // _L
    gsub = rsub * grow
    acc_rows = C + 1
    seg = _L * _MAX_ID

    mesh = plsc.VectorSubcoreMesh(core_axis_name="c", subcore_axis_name="s")

    @functools.partial(
        pl.kernel,
        out_type=jax.ShapeDtypeStruct((_NW, acc_rows, _MAX_ID), jnp.float32),
        mesh=mesh,
        compiler_params=pltpu.CompilerParams(needs_layout_passes=False),
        scratch_types=[
            pltpu.VMEM((rows, W), jnp.int32),
            pltpu.VMEM((nbuf, C, rsub, W), jnp.float32),
            pltpu.VMEM((acc_rows * seg,), jnp.float32),
            pltpu.VMEM((acc_rows, _MAX_ID), jnp.float32),
            pltpu.SemaphoreType.DMA,
            pltpu.SemaphoreType.DMA,
            pltpu.SemaphoreType.DMA,
        ],
    )
    def sc_kernel(emb_hbm, ids_hbm, out_hbm, ids_v, val_v, acc_v, stage_v,
                  sem_ids, sem_a, sem_b):
        wid = lax.axis_index("s") * 2 + lax.axis_index("c")
        frame = wid // tiles_per_frame
        b = frame // F
        f = frame % F
        r0 = (wid % tiles_per_frame) * rows

        pltpu.make_async_copy(
            ids_hbm.at[b, f, pl.ds(r0, rows), :], ids_v, sem_ids).start()
        sems = (sem_a, sem_b)
        for buf in range(nbuf - 1):
            pltpu.make_async_copy(
                emb_hbm.at[b, f, :, pl.ds(r0 + buf * rsub, rsub), :],
                val_v.at[buf], sems[buf]).start()

        zero = jnp.zeros((_L,), jnp.float32)

        def zero_body(i, carry):
            acc_v[pl.ds(i * _L, _L)] = zero
            return carry

        lax.fori_loop(0, acc_rows * seg // _L, zero_body, 0, unroll=8)
        pltpu.make_async_copy(
            ids_hbm.at[b, f, pl.ds(r0, rows), :], ids_v, sem_ids).wait()

        lane_off = lax.iota(jnp.int32, _L) * _MAX_ID
        ones = jnp.ones((_L,), jnp.float32)

        def sub_body(i, carry):
            for buf in range(nbuf):
                s = nbuf * i + buf
                nxt = (buf + nbuf - 1) % nbuf

                @pl.when(s + nbuf - 1 < n_sub)
                def _prefetch(s=s, nxt=nxt):
                    pltpu.make_async_copy(
                        emb_hbm.at[b, f, :,
                                   pl.ds(r0 + (s + nbuf - 1) * rsub, rsub),
                                   :],
                        val_v.at[nxt], sems[nxt]).start()

                pltpu.make_async_copy(
                    emb_hbm.at[b, f, :, pl.ds(r0 + s * rsub, rsub), :],
                    val_v.at[buf], sems[buf]).wait()

                @plsc.parallel_loop(0, gsub)
                def gbody(g, s=s, buf=buf):
                    gr = g // grow
                    col = (g % grow) * _L
                    idx = ids_v[s * rsub + gr, pl.ds(col, _L)] + lane_off
                    for c in range(C):
                        plsc.addupdate_scatter(
                            acc_v, [idx + c * seg],
                            val_v[buf, c, gr, pl.ds(col, _L)])
                    plsc.addupdate_scatter(acc_v, [idx + C * seg], ones)
            return carry

        lax.fori_loop(0, n_sub // nbuf, sub_body, 0)

        def red_body(r, carry):
            base = r * seg
            for blk in range(_MAX_ID // _L):
                o = blk * _L
                vs = [acc_v[pl.ds(base + l * _MAX_ID + o, _L)]
                      for l in range(_L)]
                while len(vs) > 1:
                    vs = [vs[i] + vs[i + 1] for i in range(0, len(vs), 2)]
                stage_v[r, pl.ds(o, _L)] = vs[0]
            return carry

        lax.fori_loop(0, acc_rows, red_body, 0)

        pltpu.sync_copy(stage_v, out_hbm.at[wid])

    return sc_kernel(emb, ids)


def _tc_finalize(partials, B, F, C):
    tiles_per_frame = _NW // (B * F)

    def tc_kernel(p_ref, o_ref):
        p = p_ref[...]
        p = p.reshape(B * F, tiles_per_frame, C + 1, _MAX_ID).sum(axis=1)
        sums = p[:, :C, :].reshape(B, F, C, _MAX_ID)
        counts = p[:, C, :].reshape(B, F, _MAX_ID)
        means = sums / jnp.maximum(counts, 1.0)[:, :, None, :]
        idpos = lax.broadcasted_iota(jnp.int32, (B, F, _MAX_ID), 2) > 0
        present = (counts > 0.0) & idpos
        common = present[:, :-1] & present[:, 1:]
        d = means[:, 1:] - means[:, :-1]
        dist = jnp.sum(d * d, axis=2)
        total = jnp.sum(jnp.where(common, dist, 0.0))
        valid = jnp.sum(common.astype(jnp.float32))
        o_ref[0, 0] = jnp.where(valid > 0.0,
                                total / jnp.maximum(valid, 1.0),
                                jnp.float32(0.0))

    return pl.pallas_call(
        tc_kernel,
        out_shape=jax.ShapeDtypeStruct((1, 1), jnp.float32),
        out_specs=pl.BlockSpec(memory_space=pltpu.SMEM),
    )(partials)


def kernel(embeddings, track_ids):
    B, F, C, H, W = embeddings.shape
    ids = track_ids.reshape(B, F, H, W).astype(jnp.int32)
    partials = _sc_partials(embeddings, ids)
    return _tc_finalize(partials, B, F, C)[0, 0]
